# baseline (speedup 1.0000x reference)
import functools

import jax
import jax.numpy as jnp
from jax.experimental import pallas as pl
from jax.experimental.pallas import tpu as pltpu

E = 16
U = 1024
V = 1024
Z = 8192

TM = 512
MB = Z // TM
NCHUNK = 4
CE = E // NCHUNK
PC = MB + CE - 1


def _gemm_first(meta_ref, x_ref, w_ref, o_ref):
    p = pl.program_id(0)
    start = meta_ref[2, p]
    end = meta_ref[3, p]
    first = meta_ref[4, p]
    valid = meta_ref[5, p]

    @pl.when(valid == 1)
    def _():
        row = jax.lax.broadcasted_iota(jnp.int32, (TM, 1), 0)
        mask = ((row >= start) & (row < end)).astype(jnp.float32)
        xm = x_ref[...] * mask
        acc = jnp.dot(xm, w_ref[0], preferred_element_type=jnp.float32)

        @pl.when(first == 1)
        def _():
            o_ref[...] = acc

        @pl.when(first == 0)
        def _():
            o_ref[...] += acc


def _gemm_chain(meta_ref, x_ref, w_ref, prev_ref, o_ref):
    p = pl.program_id(0)
    start = meta_ref[2, p]
    end = meta_ref[3, p]
    first = meta_ref[4, p]
    valid = meta_ref[5, p]
    cont = meta_ref[6, p]

    @pl.when(valid == 1)
    def _():
        row = jax.lax.broadcasted_iota(jnp.int32, (TM, 1), 0)
        mask = ((row >= start) & (row < end)).astype(jnp.float32)
        xm = x_ref[...] * mask
        acc = jnp.dot(xm, w_ref[0], preferred_element_type=jnp.float32)

        @pl.when((first == 1) & (cont == 1))
        def _():
            o_ref[...] = prev_ref[...] + acc

        @pl.when((first == 1) & (cont == 0))
        def _():
            o_ref[...] = acc

        @pl.when(first == 0)
        def _():
            o_ref[...] += acc


def _chunk_metadata(seg, lo, hi, e0):
    mrange = jnp.arange(MB, dtype=jnp.int32)
    erange = jnp.arange(E + 1, dtype=jnp.int32)
    lo_c = jnp.maximum(lo, e0)
    hi_c = jnp.minimum(hi, e0 + CE - 1)
    span = jnp.maximum(hi_c - lo_c + 1, 0)
    offs = jnp.concatenate(
        [jnp.zeros((1,), jnp.int32), jnp.cumsum(span)]).astype(jnp.int32)
    total = offs[MB]
    p = jnp.arange(PC, dtype=jnp.int32)
    q = jnp.minimum(p, total - 1)
    m = (jnp.sum(offs[None, :MB] <= q[:, None], axis=1) - 1).astype(jnp.int32)
    valid = ((p < total) & (m >= 0)).astype(jnp.int32)
    mc = jnp.clip(m, 0, MB - 1)
    onehot_m = (mc[:, None] == mrange[None, :]).astype(jnp.int32)
    lo_m = jnp.sum(onehot_m * lo_c[None, :], axis=1)
    offs_m = jnp.sum(onehot_m * offs[None, :MB], axis=1)
    e = jnp.clip(lo_m + q - offs_m, e0, e0 + CE - 1)
    onehot_e = (e[:, None] == erange[None, :]).astype(jnp.int32)
    seg_e = jnp.sum(onehot_e * seg[None, :], axis=1)
    onehot_e1 = ((e + 1)[:, None] == erange[None, :]).astype(jnp.int32)
    seg_e1 = jnp.sum(onehot_e1 * seg[None, :], axis=1)
    start = jnp.clip(seg_e - mc * TM, 0, TM)
    end = jnp.clip(seg_e1 - mc * TM, 0, TM)
    firstf = ((p == offs_m) & (valid == 1)).astype(jnp.int32)
    lo_at_m = jnp.sum(onehot_m * lo[None, :], axis=1)
    cont = (lo_at_m < e0).astype(jnp.int32)
    return jnp.stack([mc, e - e0, start, end, firstf, valid, cont])


def _segments(ids):
    ids = ids.astype(jnp.int32)
    erange = jnp.arange(E + 1, dtype=jnp.int32)
    seg = jnp.sum(ids[None, :] < erange[:, None], axis=1).astype(jnp.int32)
    mrange = jnp.arange(MB, dtype=jnp.int32)
    first_tok = mrange * TM
    last_tok = first_tok + (TM - 1)
    lo = (jnp.sum(seg[None, :] <= first_tok[:, None], axis=1) - 1).astype(jnp.int32)
    hi = (jnp.sum(seg[None, :] <= last_tok[:, None], axis=1) - 1).astype(jnp.int32)
    return seg, lo, hi


def _first_specs():
    return dict(
        in_specs=[
            pl.BlockSpec((TM, U), lambda p, meta: (meta[0, p], 0)),
            pl.BlockSpec((1, U, V), lambda p, meta: (meta[1, p], 0, 0)),
        ],
        out_specs=pl.BlockSpec((TM, V), lambda p, meta: (meta[0, p], 0)),
    )


@jax.jit
def kernel(weights, x, expert_ids):
    seg, lo, hi = _segments(expert_ids)

    specs = _first_specs()
    grid_spec0 = pltpu.PrefetchScalarGridSpec(
        num_scalar_prefetch=1, grid=(PC,), **specs)
    cparams = pltpu.CompilerParams(dimension_semantics=("arbitrary",))

    w0 = weights[0:CE].reshape(CE, U, V)
    meta0 = _chunk_metadata(seg, lo, hi, 0)
    out = pl.pallas_call(
        _gemm_first,
        grid_spec=grid_spec0,
        out_shape=jax.ShapeDtypeStruct((Z, V), jnp.float32),
        compiler_params=cparams,
    )(meta0, x, w0)

    chain_specs = dict(
        in_specs=specs["in_specs"]
        + [pl.BlockSpec((TM, V), lambda p, meta: (meta[0, p], 0))],
        out_specs=specs["out_specs"],
    )
    grid_spec1 = pltpu.PrefetchScalarGridSpec(
        num_scalar_prefetch=1, grid=(PC,), **chain_specs)
    for c in range(1, NCHUNK):
        wc = weights[c * CE:(c + 1) * CE].reshape(CE, U, V)
        metac = _chunk_metadata(seg, lo, hi, c * CE)
        out = pl.pallas_call(
            _gemm_chain,
            grid_spec=grid_spec1,
            out_shape=jax.ShapeDtypeStruct((Z, V), jnp.float32),
            input_output_aliases={3: 0},
            compiler_params=cparams,
        )(metac, x, wc, out)
    return out


# speedup vs baseline: 1.5002x; 1.5002x over previous
import functools

import jax
import jax.numpy as jnp
from jax.experimental import pallas as pl
from jax.experimental.pallas import tpu as pltpu

E = 16
U = 1024
V = 1024
Z = 8192

TM = 512
MB = Z // TM
P = MB + E - 1


def _gemm_body(meta_ref, x_ref, w_ref, o_ref):
    p = pl.program_id(0)
    start = meta_ref[2, p]
    end = meta_ref[3, p]
    first = meta_ref[4, p]
    valid = meta_ref[5, p]

    @pl.when(valid == 1)
    def _():
        row = jax.lax.broadcasted_iota(jnp.int32, (TM, 1), 0)
        mask = ((row >= start) & (row < end)).astype(jnp.float32)
        xm = x_ref[...] * mask
        acc = jnp.dot(xm, w_ref[0], preferred_element_type=jnp.float32)

        @pl.when(first == 1)
        def _():
            o_ref[...] = acc

        @pl.when(first == 0)
        def _():
            o_ref[...] += acc


def _pair_metadata(ids):
    ids = ids.astype(jnp.int32)
    erange = jnp.arange(E + 1, dtype=jnp.int32)
    seg = jnp.sum(ids[None, :] < erange[:, None], axis=1).astype(jnp.int32)
    mrange = jnp.arange(MB, dtype=jnp.int32)
    first_tok = mrange * TM
    last_tok = first_tok + (TM - 1)
    lo = (jnp.sum(seg[None, :] <= first_tok[:, None], axis=1) - 1).astype(jnp.int32)
    hi = (jnp.sum(seg[None, :] <= last_tok[:, None], axis=1) - 1).astype(jnp.int32)
    span = hi - lo + 1
    offs = jnp.concatenate(
        [jnp.zeros((1,), jnp.int32), jnp.cumsum(span)]).astype(jnp.int32)
    total = offs[MB]
    p = jnp.arange(P, dtype=jnp.int32)
    q = jnp.minimum(p, total - 1)
    m = (jnp.sum(offs[None, :MB] <= q[:, None], axis=1) - 1).astype(jnp.int32)
    onehot_m = (m[:, None] == mrange[None, :]).astype(jnp.int32)
    lo_m = jnp.sum(onehot_m * lo[None, :], axis=1)
    offs_m = jnp.sum(onehot_m * offs[None, :MB], axis=1)
    e = lo_m + q - offs_m
    onehot_e = (e[:, None] == erange[None, :]).astype(jnp.int32)
    seg_e = jnp.sum(onehot_e * seg[None, :], axis=1)
    onehot_e1 = ((e + 1)[:, None] == erange[None, :]).astype(jnp.int32)
    seg_e1 = jnp.sum(onehot_e1 * seg[None, :], axis=1)
    start = jnp.clip(seg_e - m * TM, 0, TM)
    end = jnp.clip(seg_e1 - m * TM, 0, TM)
    valid = (p < total).astype(jnp.int32)
    firstf = ((p == offs_m) & (p < total)).astype(jnp.int32)
    return jnp.stack([jnp.clip(m, 0, MB - 1), jnp.clip(e, 0, E - 1),
                      start, end, firstf, valid])


@jax.jit
def kernel(weights, x, expert_ids):
    meta = _pair_metadata(expert_ids)
    wr = weights.reshape(E, U, V)
    grid_spec = pltpu.PrefetchScalarGridSpec(
        num_scalar_prefetch=1,
        grid=(P,),
        in_specs=[
            pl.BlockSpec((TM, U), lambda p, meta: (meta[0, p], 0)),
            pl.BlockSpec((1, U, V), lambda p, meta: (meta[1, p], 0, 0)),
        ],
        out_specs=pl.BlockSpec((TM, V), lambda p, meta: (meta[0, p], 0)),
    )
    out = pl.pallas_call(
        _gemm_body,
        grid_spec=grid_spec,
        out_shape=jax.ShapeDtypeStruct((Z, V), jnp.float32),
        compiler_params=pltpu.CompilerParams(
            dimension_semantics=("arbitrary",),
        ),
    )(meta, x, wr)
    return out
